# Initial kernel scaffold; baseline (speedup 1.0000x reference)
import jax
import jax.numpy as jnp
from jax.experimental import pallas as pl


def kernel(pred_raw, F_ext, elem_lengths, prop_E, prop_A, prop_I22, elem_directions, u_c, theta_c, F_c, connectivity):
    raise NotImplementedError("write your pallas kernel here")


# speedup vs baseline: 11.5628x; 11.5628x over previous
import functools

import jax
import jax.numpy as jnp
from jax import lax
from jax.experimental import pallas as pl
from jax.experimental.pallas import tpu as pltpu
from jax.experimental.pallas import tpu_sc as plsc

_N_NODES = 50000
_N_ELEM = 800000
_NC = 2
_NS = 16
_NW = _NC * _NS
_EPW = _N_ELEM // _NW
_CHUNK = 1000
_NCHUNK = _EPW // _CHUNK
_NFULL = _CHUNK // 16
_TAIL = _CHUNK - _NFULL * 16
_NGRP = _NFULL + 1
_SEC = _NGRP * 16


def _sc_body(u_ref, na_ref, nb_ref, c_ref, s_ref, len_ref, pe_ref, pa_ref,
             pi_ref, out_ref,
             na_v, nb_v, gidx_v, rows_v, c_v, s_v, len_v, pe_v, pa_v, pi_v,
             acc_v, sem_n, sem_g, sem_l):
    cid = lax.axis_index("c")
    sid = lax.axis_index("s")
    wid = sid * _NC + cid
    iota = lax.iota(jnp.int32, 16)
    izero = iota * 0

    na_v[pl.ds(_SEC - 16, 16)] = izero
    nb_v[pl.ds(_SEC - 16, 16)] = izero

    def build(g, _):
        off = g * 16
        a3 = na_v[pl.ds(off, 16)] * 3
        gidx_v[pl.ds(off, 16)] = a3
        gidx_v[pl.ds(_SEC + off, 16)] = a3 + 1
        gidx_v[pl.ds(2 * _SEC + off, 16)] = a3 + 2
        b3 = nb_v[pl.ds(off, 16)] * 3
        gidx_v[pl.ds(3 * _SEC + off, 16)] = b3
        gidx_v[pl.ds(4 * _SEC + off, 16)] = b3 + 1
        gidx_v[pl.ds(5 * _SEC + off, 16)] = b3 + 2
        return _

    def group(g, acc, mask):
        off = g * 16
        xA = rows_v[pl.ds(off, 16)]
        yA = rows_v[pl.ds(_SEC + off, 16)]
        tA = rows_v[pl.ds(2 * _SEC + off, 16)]
        xB = rows_v[pl.ds(3 * _SEC + off, 16)]
        yB = rows_v[pl.ds(4 * _SEC + off, 16)]
        tB = rows_v[pl.ds(5 * _SEC + off, 16)]
        c = c_v[pl.ds(off, 16)]
        s = s_v[pl.ds(off, 16)]
        u1 = c * xA + s * yA
        v1 = c * yA - s * xA
        u2 = c * xB + s * yB
        v2 = c * yB - s * xB
        du = u1 - u2
        dv = v1 - v2
        ts = tA + tB
        lv = len_v[pl.ds(off, 16)]
        eav = pe_v[pl.ds(off, 16)] * pa_v[pl.ds(off, 16)]
        eiv = pe_v[pl.ds(off, 16)] * pi_v[pl.ds(off, 16)]
        rL = 1.0 / lv
        ea_L = eav * rL
        ei_L = eiv * rL
        ei_L2 = ei_L * rL
        ei_L3 = ei_L2 * rL
        ue = 0.5 * (ea_L * du * du + 12.0 * ei_L3 * dv * dv
                    + 12.0 * ei_L2 * dv * ts
                    + 4.0 * ei_L * (tA * tA + tB * tB + tA * tB))
        if mask is not None:
            ue = jnp.where(mask, ue, 0.0)
        return acc + ue

    def chunk(k, acc):
        base = wid * _EPW + k * _CHUNK
        dn1 = pltpu.async_copy(na_ref.at[pl.ds(base, _CHUNK)],
                               na_v.at[pl.ds(0, _CHUNK)], sem_n)
        dn2 = pltpu.async_copy(nb_ref.at[pl.ds(base, _CHUNK)],
                               nb_v.at[pl.ds(0, _CHUNK)], sem_n)
        d1 = pltpu.async_copy(c_ref.at[pl.ds(base, _CHUNK)],
                              c_v.at[pl.ds(0, _CHUNK)], sem_l)
        d2 = pltpu.async_copy(s_ref.at[pl.ds(base, _CHUNK)],
                              s_v.at[pl.ds(0, _CHUNK)], sem_l)
        d3 = pltpu.async_copy(len_ref.at[pl.ds(base, _CHUNK)],
                              len_v.at[pl.ds(0, _CHUNK)], sem_l)
        d4 = pltpu.async_copy(pe_ref.at[pl.ds(base, _CHUNK)],
                              pe_v.at[pl.ds(0, _CHUNK)], sem_l)
        d5 = pltpu.async_copy(pa_ref.at[pl.ds(base, _CHUNK)],
                              pa_v.at[pl.ds(0, _CHUNK)], sem_l)
        d6 = pltpu.async_copy(pi_ref.at[pl.ds(base, _CHUNK)],
                              pi_v.at[pl.ds(0, _CHUNK)], sem_l)
        dn1.wait()
        dn2.wait()
        lax.fori_loop(0, _NGRP, build, 0)
        dg = pltpu.async_copy(u_ref.at[gidx_v], rows_v, sem_g)
        dg.wait()
        d1.wait()
        d2.wait()
        d3.wait()
        d4.wait()
        d5.wait()
        d6.wait()
        acc = lax.fori_loop(0, _NFULL, lambda g, a: group(g, a, None), acc)
        acc = group(_NFULL, acc, iota < _TAIL)
        return acc

    acc = lax.fori_loop(0, _NCHUNK, chunk, jnp.zeros((16,), jnp.float32))
    acc_v[...] = acc
    pltpu.sync_copy(acc_v, out_ref.at[wid])


_sc_energy = functools.partial(
    pl.kernel,
    out_type=jax.ShapeDtypeStruct((_NW, 16), jnp.float32),
    mesh=plsc.VectorSubcoreMesh(core_axis_name="c", subcore_axis_name="s"),
    scratch_types=[
        pltpu.VMEM((_SEC,), jnp.int32),
        pltpu.VMEM((_SEC,), jnp.int32),
        pltpu.VMEM((6 * _SEC,), jnp.int32),
        pltpu.VMEM((6 * _SEC,), jnp.float32),
        pltpu.VMEM((_SEC,), jnp.float32),
        pltpu.VMEM((_SEC,), jnp.float32),
        pltpu.VMEM((_SEC,), jnp.float32),
        pltpu.VMEM((_SEC,), jnp.float32),
        pltpu.VMEM((_SEC,), jnp.float32),
        pltpu.VMEM((_SEC,), jnp.float32),
        pltpu.VMEM((16,), jnp.float32),
        pltpu.SemaphoreType.DMA,
        pltpu.SemaphoreType.DMA,
        pltpu.SemaphoreType.DMA,
    ],
)(_sc_body)


_ROWS_BLK = 2000
_GRID_A = _N_NODES // _ROWS_BLK


def _tc_a_body(uc_ref, tc_ref, pred_ref, fext_ref, uph_ref, w_ref):
    i = pl.program_id(0)
    col = lax.broadcasted_iota(jnp.int32, (_ROWS_BLK, 3), 1)
    scale = jnp.where(col == 2, tc_ref[0], uc_ref[0])
    u = pred_ref[...] * scale
    uph_ref[...] = u

    @pl.when(i == 0)
    def _():
        w_ref[0] = 0.0

    w_ref[0] += jnp.sum(fext_ref[...] * u)


def _tc_a(pred_raw, F_ext, u_c, theta_c):
    return pl.pallas_call(
        _tc_a_body,
        grid=(_GRID_A,),
        in_specs=[
            pl.BlockSpec(memory_space=pltpu.SMEM),
            pl.BlockSpec(memory_space=pltpu.SMEM),
            pl.BlockSpec((_ROWS_BLK, 3), lambda i: (i, 0)),
            pl.BlockSpec((_ROWS_BLK, 3), lambda i: (i, 0)),
        ],
        out_specs=[
            pl.BlockSpec((_ROWS_BLK, 3), lambda i: (i, 0)),
            pl.BlockSpec(memory_space=pltpu.SMEM),
        ],
        out_shape=[
            jax.ShapeDtypeStruct((_N_NODES, 3), jnp.float32),
            jax.ShapeDtypeStruct((1,), jnp.float32),
        ],
    )(u_c, theta_c, pred_raw, F_ext)


def _tc_b_body(w_ref, fc_ref, uc_ref, parts_ref, out_ref):
    u_int = jnp.sum(parts_ref[...])
    pi = u_int - w_ref[0]
    e_c = jnp.maximum(fc_ref[0] * uc_ref[0], 1e-30)
    out_ref[0] = pi / e_c


def _tc_b(parts, w, F_c, u_c):
    return pl.pallas_call(
        _tc_b_body,
        in_specs=[
            pl.BlockSpec(memory_space=pltpu.SMEM),
            pl.BlockSpec(memory_space=pltpu.SMEM),
            pl.BlockSpec(memory_space=pltpu.SMEM),
            pl.BlockSpec((_NW, 16), lambda: (0, 0)),
        ],
        out_specs=pl.BlockSpec(memory_space=pltpu.SMEM),
        out_shape=jax.ShapeDtypeStruct((1,), jnp.float32),
    )(w, F_c, u_c, parts)


def kernel(pred_raw, F_ext, elem_lengths, prop_E, prop_A, prop_I22,
           elem_directions, u_c, theta_c, F_c, connectivity):
    u_phys, w_part = _tc_a(pred_raw, F_ext, u_c, theta_c)
    parts = _sc_energy(
        u_phys.reshape(-1),
        connectivity[:, 0],
        connectivity[:, 1],
        elem_directions[:, 0],
        elem_directions[:, 2],
        elem_lengths, prop_E, prop_A, prop_I22,
    )
    pi_norm = _tc_b(parts, w_part, F_c, u_c)
    return (pi_norm, pred_raw, u_phys)
